# baseline (speedup 1.0000x reference)
import functools

import jax
import jax.numpy as jnp
from jax import lax
from jax.experimental import pallas as pl
from jax.experimental.pallas import tpu as pltpu
from jax.experimental.pallas import tpu_sc as plsc

_B = 16384
_D = 32
_NC = 2
_NS = 16
_NW = _NC * _NS
_BPW = _B // _NW
_CH = 16
_NCH = _BPW // _CH


def _cf_body(uidx_hbm, iidx_hbm, utab_hbm, itab_hbm, ubias_hbm, ibias_hbm,
             out_hbm, uidx_v, iidx_v, ucols_v, icols_v, ub_v, ib_v, out_v,
             sem_u, sem_i, sem_bu, sem_bi):
    wid = lax.axis_index("s") * _NC + lax.axis_index("c")
    base = wid * _BPW

    pltpu.sync_copy(uidx_hbm.at[pl.ds(base, _BPW)], uidx_v)
    pltpu.sync_copy(iidx_hbm.at[pl.ds(base, _BPW)], iidx_v)

    for d in range(_D):
        pltpu.async_copy(utab_hbm.at[d].at[uidx_v], ucols_v.at[d], sem_u)
        pltpu.async_copy(itab_hbm.at[d].at[iidx_v], icols_v.at[d], sem_i)
    cbu = pltpu.async_copy(ubias_hbm.at[uidx_v], ub_v, sem_bu)
    cbi = pltpu.async_copy(ibias_hbm.at[iidx_v], ib_v, sem_bi)

    pltpu.make_async_copy(utab_hbm.at[:, pl.ds(0, _BPW)], ucols_v,
                          sem_u).wait()
    pltpu.make_async_copy(itab_hbm.at[:, pl.ds(0, _BPW)], icols_v,
                          sem_i).wait()
    cbu.wait()
    cbi.wait()

    def chunk(c, _):
        sl = pl.ds(c * _CH, _CH)
        acc = ub_v[sl] + ib_v[sl]
        for d in range(_D):
            acc = acc + ucols_v[d, sl] * icols_v[d, sl]
        out_v[sl] = acc
        return _

    lax.fori_loop(0, _NCH, chunk, None)
    pltpu.sync_copy(out_v, out_hbm.at[pl.ds(base, _BPW)])


@jax.jit
def _cf_predict(user_indices, item_indices, user_emb_t, item_emb_t,
                user_bias, item_bias):
    mesh = plsc.VectorSubcoreMesh(core_axis_name="c", subcore_axis_name="s")
    f = pl.kernel(
        _cf_body,
        out_type=jax.ShapeDtypeStruct((_B,), jnp.float32),
        mesh=mesh,
        scratch_types=[
            pltpu.VMEM((_BPW,), jnp.int32),
            pltpu.VMEM((_BPW,), jnp.int32),
            pltpu.VMEM((_D, _BPW), jnp.float32),
            pltpu.VMEM((_D, _BPW), jnp.float32),
            pltpu.VMEM((_BPW,), jnp.float32),
            pltpu.VMEM((_BPW,), jnp.float32),
            pltpu.VMEM((_BPW,), jnp.float32),
            pltpu.SemaphoreType.DMA,
            pltpu.SemaphoreType.DMA,
            pltpu.SemaphoreType.DMA,
            pltpu.SemaphoreType.DMA,
        ],
        compiler_params=pltpu.CompilerParams(
            needs_layout_passes=False, use_tc_tiling_on_sc=False),
    )
    return f(user_indices, item_indices, user_emb_t, item_emb_t,
             user_bias, item_bias)


def kernel(user_indices, item_indices, user_emb_table, item_emb_table,
           user_bias_table, item_bias_table):
    return _cf_predict(user_indices, item_indices, user_emb_table.T,
                       item_emb_table.T, user_bias_table.reshape(-1),
                       item_bias_table.reshape(-1))
